# baseline (speedup 1.0000x reference)
import functools

import jax
import jax.numpy as jnp
from jax import lax
from jax.experimental import pallas as pl
from jax.experimental.pallas import tpu as pltpu
from jax.experimental.pallas import tpu_sc as plsc

B, T, C = 4, 8192, 1024
T_BLK = 512

NROWS = B * T
N_WORKERS = 32
WROWS = T // N_WORKERS
CR = 16
N_CHUNKS = WROWS // CR
LANES = 16


def _tc_add_kernel(x_ref, w_ref, o_ref):
    o_ref[...] = x_ref[...] + w_ref[...][None, :, :]


def _tc_kernel(x, embed_weight, nb=B):
    w = embed_weight[:T]
    return pl.pallas_call(
        _tc_add_kernel,
        grid=(T // T_BLK,),
        in_specs=[
            pl.BlockSpec((nb, T_BLK, C), lambda t: (0, t, 0)),
            pl.BlockSpec((T_BLK, C), lambda t: (t, 0)),
        ],
        out_specs=pl.BlockSpec((nb, T_BLK, C), lambda t: (0, t, 0)),
        out_shape=jax.ShapeDtypeStruct((nb, T, C), x.dtype),
    )(x, w)


def _sc_body(x_hbm, w_hbm, o_hbm,
             ob0, ob1, ob2, ob3, wb0, wb1,
             xs0, xs1, xs2, xs3, os0, os1, os2, os3, ws0, ws1):
    wid = lax.axis_index("s") * 2 + lax.axis_index("c")
    wbase = wid * WROWS

    ob = (ob0, ob1, ob2, ob3)
    wb = (wb0, wb1)
    xs = (xs0, xs1, xs2, xs3)
    osm = (os0, os1, os2, os3)
    ws = (ws0, ws1)

    def start_x(j, b, q):
        pltpu.async_copy(
            x_hbm.at[pl.ds(b * T + wbase + j * CR, CR)], ob[q], xs[q])

    def wait_x(q):
        pltpu.make_async_copy(x_hbm.at[pl.ds(0, CR)], ob[q], xs[q]).wait()

    def start_out(j, b, q):
        pltpu.async_copy(
            ob[q], o_hbm.at[pl.ds(b * T + wbase + j * CR, CR)], osm[q])

    def wait_out(q):
        pltpu.make_async_copy(ob[q], o_hbm.at[pl.ds(0, CR)], osm[q]).wait()

    def start_w(j, s):
        pltpu.async_copy(w_hbm.at[pl.ds(wbase + j * CR, CR)], wb[s], ws[s])

    def wait_w(s):
        pltpu.make_async_copy(w_hbm.at[pl.ds(0, CR)], wb[s], ws[s]).wait()

    start_w(0, 0)
    start_x(0, 0, 0)
    start_x(0, 1, 1)

    @pl.loop(0, N_CHUNKS, step=2)
    def _chunks(j0):
        for dj in range(2):
            j = j0 + dj
            wslot = dj
            wait_w(wslot)
            pl.when(j + 1 < N_CHUNKS)(lambda: start_w(j + 1, 1 - wslot))
            for b in range(B):
                q = b
                i = j * B + b
                wait_x(q)

                def add(s):
                    r = s // (C // LANES)
                    sl = pl.ds((s % (C // LANES)) * LANES, LANES)
                    plsc.addupdate(ob[q].at[r, sl], wb[wslot][r, sl])

                plsc.parallel_loop(0, CR * C // LANES, 1, unroll=8)(add)
                start_out(j, b, q)
                nq = (b + 2) % 4
                if b < 2:
                    pl.when(i >= 2)(lambda: wait_out(nq))
                    start_x(j, b + 2, nq)
                else:
                    def _next():
                        wait_out(nq)
                        start_x(j + 1, b - 2, nq)
                    pl.when(j + 1 < N_CHUNKS)(_next)

    wait_out(0)
    wait_out(1)
    wait_out(2)
    wait_out(3)


def _sc_body_1b(x_hbm, w_hbm, o_hbm,
                ob0, ob1, ob2, ob3, wb0, wb1,
                xs0, xs1, xs2, xs3, os0, os1, os2, os3, ws0, ws1):
    wid = lax.axis_index("s") * 2 + lax.axis_index("c")
    wbase = wid * WROWS

    ob = (ob0, ob1, ob2, ob3)
    wb = (wb0, wb1)
    xs = (xs0, xs1, xs2, xs3)
    osm = (os0, os1, os2, os3)
    ws = (ws0, ws1)

    def start_x(j, q):
        pltpu.async_copy(
            x_hbm.at[pl.ds(XB_OFF * T + wbase + j * CR, CR)], ob[q], xs[q])

    def wait_x(q):
        pltpu.make_async_copy(x_hbm.at[pl.ds(0, CR)], ob[q], xs[q]).wait()

    def start_out(j, q):
        pltpu.async_copy(ob[q], o_hbm.at[pl.ds(wbase + j * CR, CR)], osm[q])

    def wait_out(q):
        pltpu.make_async_copy(ob[q], o_hbm.at[pl.ds(0, CR)], osm[q]).wait()

    def start_w(j, s):
        pltpu.async_copy(w_hbm.at[pl.ds(wbase + j * CR, CR)], wb[s], ws[s])

    def wait_w(s):
        pltpu.make_async_copy(w_hbm.at[pl.ds(0, CR)], wb[s], ws[s]).wait()

    start_w(0, 0)
    start_x(0, 0)
    start_x(1, 1)

    @pl.loop(0, N_CHUNKS, step=4)
    def _chunks(j0):
        for dj in range(4):
            j = j0 + dj
            q = dj
            wslot = dj % 2
            wait_w(wslot)
            pl.when(j + 1 < N_CHUNKS)(lambda: start_w(j + 1, 1 - wslot))
            wait_x(q)

            def add(s):
                r = s // (C // LANES)
                sl = pl.ds((s % (C // LANES)) * LANES, LANES)
                plsc.addupdate(ob[q].at[r, sl], wb[wslot][r, sl])

            plsc.parallel_loop(0, CR * C // LANES, 1, unroll=8)(add)
            start_out(j, q)
            nq = (dj + 2) % 4
            pl.when(jnp.logical_and(j >= 2, j + 2 < N_CHUNKS))(
                lambda: wait_out(nq))
            pl.when(j + 2 < N_CHUNKS)(lambda: start_x(j + 2, nq))

    wait_out(0)
    wait_out(1)
    wait_out(2)
    wait_out(3)


XB_OFF = 3


def _sc_kernel_1b(x, embed_weight):
    mesh = plsc.VectorSubcoreMesh(core_axis_name="c", subcore_axis_name="s")
    run = pl.kernel(
        _sc_body_1b,
        out_type=jax.ShapeDtypeStruct((T, C), jnp.float32),
        mesh=mesh,
        scratch_types=(
            [pltpu.VMEM((CR, C), jnp.float32)] * 6
            + [pltpu.SemaphoreType.DMA] * 10
        ),
        compiler_params=pltpu.CompilerParams(use_tc_tiling_on_sc=True),
    )
    return run(x.reshape(NROWS, C), embed_weight[:T])


def _hybrid_kernel(x, embed_weight):
    sc_out = _sc_kernel_1b(x, embed_weight)
    tc_out = _tc_kernel(x, embed_weight, nb=XB_OFF)
    return jnp.concatenate([tc_out, sc_out[None]], axis=0)


def _sc_kernel(x, embed_weight):
    mesh = plsc.VectorSubcoreMesh(core_axis_name="c", subcore_axis_name="s")
    run = pl.kernel(
        _sc_body,
        out_type=jax.ShapeDtypeStruct((NROWS, C), jnp.float32),
        mesh=mesh,
        scratch_types=(
            [pltpu.VMEM((CR, C), jnp.float32)] * 6
            + [pltpu.SemaphoreType.DMA] * 10
        ),
        compiler_params=pltpu.CompilerParams(use_tc_tiling_on_sc=True),
    )
    out = run(x.reshape(NROWS, C), embed_weight[:T])
    return out.reshape(B, T, C)


def kernel(x, embed_weight):
    return _hybrid_kernel(x, embed_weight)


# speedup vs baseline: 1.6592x; 1.6592x over previous
import functools

import jax
import jax.numpy as jnp
from jax import lax
from jax.experimental import pallas as pl
from jax.experimental.pallas import tpu as pltpu
from jax.experimental.pallas import tpu_sc as plsc

B, T, C = 4, 8192, 1024
T_BLK = 512

NROWS = B * T
N_WORKERS = 32
WROWS = T // N_WORKERS
CR = 16
N_CHUNKS = WROWS // CR
LANES = 16


def _tc_add_kernel(x_ref, w_ref, o_ref):
    o_ref[...] = x_ref[...] + w_ref[...][None, :, :]


def _tc_kernel(x, embed_weight, nb=B):
    w = embed_weight[:T]
    return pl.pallas_call(
        _tc_add_kernel,
        grid=(T // T_BLK,),
        in_specs=[
            pl.BlockSpec((nb, T_BLK, C), lambda t: (0, t, 0)),
            pl.BlockSpec((T_BLK, C), lambda t: (t, 0)),
        ],
        out_specs=pl.BlockSpec((nb, T_BLK, C), lambda t: (0, t, 0)),
        out_shape=jax.ShapeDtypeStruct((nb, T, C), x.dtype),
    )(x, w)


def _sc_body(x_hbm, w_hbm, o_hbm,
             ob0, ob1, ob2, ob3, wb0, wb1,
             xs0, xs1, xs2, xs3, os0, os1, os2, os3, ws0, ws1):
    wid = lax.axis_index("s") * 2 + lax.axis_index("c")
    wbase = wid * WROWS

    ob = (ob0, ob1, ob2, ob3)
    wb = (wb0, wb1)
    xs = (xs0, xs1, xs2, xs3)
    osm = (os0, os1, os2, os3)
    ws = (ws0, ws1)

    def start_x(j, b, q):
        pltpu.async_copy(
            x_hbm.at[pl.ds(b * T + wbase + j * CR, CR)], ob[q], xs[q])

    def wait_x(q):
        pltpu.make_async_copy(x_hbm.at[pl.ds(0, CR)], ob[q], xs[q]).wait()

    def start_out(j, b, q):
        pltpu.async_copy(
            ob[q], o_hbm.at[pl.ds(b * T + wbase + j * CR, CR)], osm[q])

    def wait_out(q):
        pltpu.make_async_copy(ob[q], o_hbm.at[pl.ds(0, CR)], osm[q]).wait()

    def start_w(j, s):
        pltpu.async_copy(w_hbm.at[pl.ds(wbase + j * CR, CR)], wb[s], ws[s])

    def wait_w(s):
        pltpu.make_async_copy(w_hbm.at[pl.ds(0, CR)], wb[s], ws[s]).wait()

    start_w(0, 0)
    start_x(0, 0, 0)
    start_x(0, 1, 1)

    @pl.loop(0, N_CHUNKS, step=2)
    def _chunks(j0):
        for dj in range(2):
            j = j0 + dj
            wslot = dj
            wait_w(wslot)
            pl.when(j + 1 < N_CHUNKS)(lambda: start_w(j + 1, 1 - wslot))
            for b in range(B):
                q = b
                i = j * B + b
                wait_x(q)
                nq = (b + 2) % 4
                if b < 2:
                    pl.when(i >= 2)(lambda: wait_out(nq))
                    start_x(j, b + 2, nq)
                else:
                    def _next():
                        wait_out(nq)
                        start_x(j + 1, b - 2, nq)
                    pl.when(j + 1 < N_CHUNKS)(_next)

                def add(s):
                    r = s // (C // LANES)
                    sl = pl.ds((s % (C // LANES)) * LANES, LANES)
                    plsc.addupdate(ob[q].at[r, sl], wb[wslot][r, sl])

                plsc.parallel_loop(0, CR * C // LANES, 1, unroll=8)(add)
                start_out(j, b, q)

    wait_out(0)
    wait_out(1)
    wait_out(2)
    wait_out(3)


def _sc_body_1b(x_hbm, w_hbm, o_hbm,
                ob0, ob1, ob2, ob3, wb0, wb1,
                xs0, xs1, xs2, xs3, os0, os1, os2, os3, ws0, ws1):
    wid = lax.axis_index("s") * 2 + lax.axis_index("c")
    wbase = wid * WROWS

    ob = (ob0, ob1, ob2, ob3)
    wb = (wb0, wb1)
    xs = (xs0, xs1, xs2, xs3)
    osm = (os0, os1, os2, os3)
    ws = (ws0, ws1)

    def start_x(j, q):
        pltpu.async_copy(
            x_hbm.at[pl.ds(XB_OFF * T + wbase + j * CR, CR)], ob[q], xs[q])

    def wait_x(q):
        pltpu.make_async_copy(x_hbm.at[pl.ds(0, CR)], ob[q], xs[q]).wait()

    def start_out(j, q):
        pltpu.async_copy(ob[q], o_hbm.at[pl.ds(wbase + j * CR, CR)], osm[q])

    def wait_out(q):
        pltpu.make_async_copy(ob[q], o_hbm.at[pl.ds(0, CR)], osm[q]).wait()

    def start_w(j, s):
        pltpu.async_copy(w_hbm.at[pl.ds(wbase + j * CR, CR)], wb[s], ws[s])

    def wait_w(s):
        pltpu.make_async_copy(w_hbm.at[pl.ds(0, CR)], wb[s], ws[s]).wait()

    start_w(0, 0)
    start_x(0, 0)
    start_x(1, 1)

    @pl.loop(0, N_CHUNKS, step=4)
    def _chunks(j0):
        for dj in range(4):
            j = j0 + dj
            q = dj
            wslot = dj % 2
            wait_w(wslot)
            pl.when(j + 1 < N_CHUNKS)(lambda: start_w(j + 1, 1 - wslot))
            wait_x(q)

            def add(s):
                r = s // (C // LANES)
                sl = pl.ds((s % (C // LANES)) * LANES, LANES)
                plsc.addupdate(ob[q].at[r, sl], wb[wslot][r, sl])

            plsc.parallel_loop(0, CR * C // LANES, 1, unroll=8)(add)
            start_out(j, q)
            nq = (dj + 2) % 4
            pl.when(jnp.logical_and(j >= 2, j + 2 < N_CHUNKS))(
                lambda: wait_out(nq))
            pl.when(j + 2 < N_CHUNKS)(lambda: start_x(j + 2, nq))

    wait_out(0)
    wait_out(1)
    wait_out(2)
    wait_out(3)


XB_OFF = 3


def _sc_kernel_1b(x, embed_weight):
    mesh = plsc.VectorSubcoreMesh(core_axis_name="c", subcore_axis_name="s")
    run = pl.kernel(
        _sc_body_1b,
        out_type=jax.ShapeDtypeStruct((T, C), jnp.float32),
        mesh=mesh,
        scratch_types=(
            [pltpu.VMEM((CR, C), jnp.float32)] * 6
            + [pltpu.SemaphoreType.DMA] * 10
        ),
        compiler_params=pltpu.CompilerParams(use_tc_tiling_on_sc=True),
    )
    return run(x.reshape(NROWS, C), embed_weight[:T])


def _hybrid_kernel(x, embed_weight):
    sc_out = _sc_kernel_1b(x, embed_weight)
    tc_out = _tc_kernel(x, embed_weight, nb=XB_OFF)
    return jnp.concatenate([tc_out, sc_out[None]], axis=0)


def _sc_kernel(x, embed_weight):
    mesh = plsc.VectorSubcoreMesh(core_axis_name="c", subcore_axis_name="s")
    run = pl.kernel(
        _sc_body,
        out_type=jax.ShapeDtypeStruct((NROWS, C), jnp.float32),
        mesh=mesh,
        scratch_types=(
            [pltpu.VMEM((CR, C), jnp.float32)] * 6
            + [pltpu.SemaphoreType.DMA] * 10
        ),
        compiler_params=pltpu.CompilerParams(use_tc_tiling_on_sc=True),
    )
    out = run(x.reshape(NROWS, C), embed_weight[:T])
    return out.reshape(B, T, C)


def kernel(x, embed_weight):
    return _sc_kernel(x, embed_weight)
